# Initial kernel scaffold; baseline (speedup 1.0000x reference)
import jax
import jax.numpy as jnp
from jax.experimental import pallas as pl


def kernel(x, batch, W, b, gamma, beta):
    raise NotImplementedError("write your pallas kernel here")


# speedup vs baseline: 3.3198x; 3.3198x over previous
import jax
import jax.numpy as jnp
from jax.experimental import pallas as pl
from jax.experimental.pallas import tpu as pltpu

_NN = 50000
_D = 256
_NG = 512
_R = 2000
_NB = _NN // _R
_NEG = float(jnp.finfo(jnp.float32).min)
_HI = jax.lax.Precision.HIGHEST


def _pool_kernel(ids_col_ref, ids_row_ref, x_ref, w_ref, b_ref, g_ref, be_ref,
                 o_ref, cnt_ref, sum_ref, sq_ref, max_ref):
    i = pl.program_id(0)

    @pl.when(i == 0)
    def _init():
        cnt_ref[...] = jnp.zeros_like(cnt_ref)
        sum_ref[...] = jnp.zeros_like(sum_ref)
        sq_ref[...] = jnp.zeros_like(sq_ref)
        max_ref[...] = jnp.full_like(max_ref, _NEG)

    x = x_ref[...]
    ids_col = ids_col_ref[0]
    ids_row = ids_row_ref[0]

    segs = jax.lax.broadcasted_iota(jnp.int32, (_NG, 1), 0)
    onehot = (segs == ids_row).astype(jnp.float32)

    xx = jnp.concatenate([x, x * x], axis=1)
    ssq = jax.lax.dot_general(onehot, xx, (((1,), (0,)), ((), ())),
                              precision=_HI,
                              preferred_element_type=jnp.float32)
    sum_ref[...] += ssq[:, :_D]
    sq_ref[...] += ssq[:, _D:]
    cnt_ref[...] += jnp.sum(onehot, axis=1, keepdims=True)

    m = x
    k = 1
    while k < _R:
        pm = jnp.concatenate(
            [jnp.full((k, _D), _NEG, jnp.float32), m[:-k]], axis=0)
        pid = jnp.concatenate(
            [jnp.full((k, 1), -1, jnp.int32), ids_col[:-k]], axis=0)
        m = jnp.where(pid == ids_col, jnp.maximum(m, pm), m)
        k *= 2

    nid = jnp.concatenate(
        [ids_row[:, 1:], jnp.full((1, 1), -2, jnp.int32)], axis=1)
    last = (nid != ids_row).astype(jnp.float32)
    sel = onehot * last
    mloc = jax.lax.dot_general(sel, m, (((1,), (0,)), ((), ())),
                               precision=_HI,
                               preferred_element_type=jnp.float32)
    present = jnp.sum(sel, axis=1, keepdims=True) > 0.0
    max_ref[...] = jnp.maximum(max_ref[...], jnp.where(present, mloc, _NEG))

    @pl.when(i == _NB - 1)
    def _fin():
        cnt = cnt_ref[...]
        s = sum_ref[...]
        mean = s / jnp.maximum(cnt, 1.0)
        var_sum = jnp.maximum(sq_ref[...] - s * mean, 0.0)
        denom = jnp.maximum(cnt - 1.0, 1.0)
        std = jnp.sqrt(var_sum / denom)
        pooled = jnp.concatenate([mean, max_ref[...], std], axis=1)
        h = jax.lax.dot_general(pooled, w_ref[...], (((1,), (0,)), ((), ())),
                                precision=_HI,
                                preferred_element_type=jnp.float32)
        h = h + b_ref[...]
        mu = jnp.mean(h, axis=1, keepdims=True)
        var = jnp.mean((h - mu) ** 2, axis=1, keepdims=True)
        hn = (h - mu) * jax.lax.rsqrt(var + 1e-5) * g_ref[...] + be_ref[...]
        o_ref[...] = jnp.where(hn >= 0, hn, 0.01 * hn)


def kernel(x, batch, W, b, gamma, beta):
    ids = batch.astype(jnp.int32)
    ids_col = ids.reshape(_NB, _R, 1)
    ids_row = ids.reshape(_NB, 1, _R)
    b2 = b.reshape(1, _D)
    g2 = gamma.reshape(1, _D)
    be2 = beta.reshape(1, _D)
    return pl.pallas_call(
        _pool_kernel,
        grid=(_NB,),
        in_specs=[
            pl.BlockSpec((1, _R, 1), lambda i: (i, 0, 0)),
            pl.BlockSpec((1, 1, _R), lambda i: (i, 0, 0)),
            pl.BlockSpec((_R, _D), lambda i: (i, 0)),
            pl.BlockSpec((3 * _D, _D), lambda i: (0, 0)),
            pl.BlockSpec((1, _D), lambda i: (0, 0)),
            pl.BlockSpec((1, _D), lambda i: (0, 0)),
            pl.BlockSpec((1, _D), lambda i: (0, 0)),
        ],
        out_specs=pl.BlockSpec((_NG, _D), lambda i: (0, 0)),
        out_shape=jax.ShapeDtypeStruct((_NG, _D), jnp.float32),
        scratch_shapes=[
            pltpu.VMEM((_NG, 1), jnp.float32),
            pltpu.VMEM((_NG, _D), jnp.float32),
            pltpu.VMEM((_NG, _D), jnp.float32),
            pltpu.VMEM((_NG, _D), jnp.float32),
        ],
    )(ids_col, ids_row, x, W, b2, g2, be2)


# speedup vs baseline: 7.6281x; 2.2978x over previous
import jax
import jax.numpy as jnp
from jax.experimental import pallas as pl
from jax.experimental.pallas import tpu as pltpu

_NN = 50000
_D = 256
_NG = 512
_R = 2000
_NB = _NN // _R
_NEG = float(jnp.finfo(jnp.float32).min)
_HI = jax.lax.Precision.HIGHEST


def _pool_kernel(ids_col_ref, ids_row_ref, x_ref, w_ref, b_ref, g_ref, be_ref,
                 o_ref, cnt_ref, sum_ref, sq_ref, max_ref):
    i = pl.program_id(0)

    @pl.when(i == 0)
    def _init():
        cnt_ref[...] = jnp.zeros_like(cnt_ref)
        sum_ref[...] = jnp.zeros_like(sum_ref)
        sq_ref[...] = jnp.zeros_like(sq_ref)
        max_ref[...] = jnp.full_like(max_ref, _NEG)

    x = x_ref[...]
    ids_col = ids_col_ref[0]
    ids_row = ids_row_ref[0]

    segs = jax.lax.broadcasted_iota(jnp.int32, (_NG, 1), 0)
    oh = segs == ids_row
    onehot = oh.astype(jnp.bfloat16)

    xx = jnp.concatenate(
        [x.astype(jnp.bfloat16), (x * x).astype(jnp.bfloat16)], axis=1)
    ssq = jax.lax.dot_general(onehot, xx, (((1,), (0,)), ((), ())),
                              preferred_element_type=jnp.float32)
    sum_ref[...] += ssq[:, :_D]
    sq_ref[...] += ssq[:, _D:]
    cnt_ref[...] += jnp.sum(oh.astype(jnp.float32), axis=1, keepdims=True)

    m = x.astype(jnp.bfloat16)
    k = 1
    while k < _R:
        pm = jnp.concatenate(
            [jnp.full((k, _D), _NEG, jnp.bfloat16), m[:-k]], axis=0)
        pid = jnp.concatenate(
            [jnp.full((k, 1), -1, jnp.int32), ids_col[:-k]], axis=0)
        m = jnp.where(pid == ids_col, jnp.maximum(m, pm), m)
        k *= 2

    nid = jnp.concatenate(
        [ids_row[:, 1:], jnp.full((1, 1), -2, jnp.int32)], axis=1)
    last = (nid != ids_row).astype(jnp.bfloat16)
    sel = onehot * last
    mloc = jax.lax.dot_general(sel, m, (((1,), (0,)), ((), ())),
                               preferred_element_type=jnp.float32)
    present = jnp.sum(sel.astype(jnp.float32), axis=1, keepdims=True) > 0.0
    max_ref[...] = jnp.maximum(max_ref[...], jnp.where(present, mloc, _NEG))

    @pl.when(i == _NB - 1)
    def _fin():
        cnt = cnt_ref[...]
        s = sum_ref[...]
        mean = s / jnp.maximum(cnt, 1.0)
        var_sum = jnp.maximum(sq_ref[...] - s * mean, 0.0)
        denom = jnp.maximum(cnt - 1.0, 1.0)
        std = jnp.sqrt(var_sum / denom)
        pooled = jnp.concatenate([mean, max_ref[...], std], axis=1)
        h = jax.lax.dot_general(pooled, w_ref[...], (((1,), (0,)), ((), ())),
                                preferred_element_type=jnp.float32)
        h = h + b_ref[...]
        mu = jnp.mean(h, axis=1, keepdims=True)
        var = jnp.mean((h - mu) ** 2, axis=1, keepdims=True)
        hn = (h - mu) * jax.lax.rsqrt(var + 1e-5) * g_ref[...] + be_ref[...]
        o_ref[...] = jnp.where(hn >= 0, hn, 0.01 * hn)


def kernel(x, batch, W, b, gamma, beta):
    ids = batch.astype(jnp.int32)
    ids_col = ids.reshape(_NB, _R, 1)
    ids_row = ids.reshape(_NB, 1, _R)
    b2 = b.reshape(1, _D)
    g2 = gamma.reshape(1, _D)
    be2 = beta.reshape(1, _D)
    return pl.pallas_call(
        _pool_kernel,
        grid=(_NB,),
        in_specs=[
            pl.BlockSpec((1, _R, 1), lambda i: (i, 0, 0)),
            pl.BlockSpec((1, 1, _R), lambda i: (i, 0, 0)),
            pl.BlockSpec((_R, _D), lambda i: (i, 0)),
            pl.BlockSpec((3 * _D, _D), lambda i: (0, 0)),
            pl.BlockSpec((1, _D), lambda i: (0, 0)),
            pl.BlockSpec((1, _D), lambda i: (0, 0)),
            pl.BlockSpec((1, _D), lambda i: (0, 0)),
        ],
        out_specs=pl.BlockSpec((_NG, _D), lambda i: (0, 0)),
        out_shape=jax.ShapeDtypeStruct((_NG, _D), jnp.float32),
        scratch_shapes=[
            pltpu.VMEM((_NG, 1), jnp.float32),
            pltpu.VMEM((_NG, _D), jnp.float32),
            pltpu.VMEM((_NG, _D), jnp.float32),
            pltpu.VMEM((_NG, _D), jnp.float32),
        ],
    )(ids_col, ids_row, x, W, b2, g2, be2)
